# baseline (speedup 1.0000x reference)
import functools

import jax
import jax.numpy as jnp
from jax import lax
from jax.experimental import pallas as pl
from jax.experimental.pallas import tpu as pltpu
from jax.experimental.pallas import tpu_sc as plsc

_NT = 4096
_TK = 8
_H = 4096
_HW = _H // 2
_TE = _NT * _TK
_SL = 16
_LN = 128
_NC = 2
_NS = 16
_NW = _NC * _NS
_TPW = _NT // _NW
_EPW = _TPW * _TK
_T = 2
_RPC = _T * _TK
_NCH = _TPW // _T

_mesh = plsc.VectorSubcoreMesh(core_axis_name="c", subcore_axis_name="s")


@functools.partial(
    pl.kernel,
    out_type=jax.ShapeDtypeStruct((_NT, _SL, _LN), jnp.int32),
    mesh=_mesh,
    compiler_params=pltpu.CompilerParams(
        use_tc_tiling_on_sc=False, needs_layout_passes=False),
    scratch_types=[
        pltpu.VMEM((_NCH, _RPC), jnp.int32),
        pltpu.VMEM((_NCH, _RPC, 32), jnp.bfloat16),
        pltpu.VMEM((_RPC, _SL, _LN), jnp.int32),
        pltpu.VMEM((_RPC, _SL, _LN), jnp.int32),
        pltpu.VMEM((_T, _SL, _LN), jnp.int32),
        pltpu.VMEM((_T, _SL, _LN), jnp.int32),
        pltpu.SemaphoreType.DMA,
        pltpu.SemaphoreType.DMA,
        pltpu.SemaphoreType.DMA,
        pltpu.SemaphoreType.DMA,
    ],
)
def _combine(expert_hbm, inv_hbm, w_hbm, out_hbm,
             idx_v, w_v, rows0, rows1, ob0, ob1, g0, g1, o0, o1):
  wid = lax.axis_index("s") * _NC + lax.axis_index("c")
  tbase = wid * _TPW

  pltpu.sync_copy(inv_hbm.at[wid], idx_v)
  pltpu.sync_copy(w_hbm.at[wid], w_v)

  rows = (rows0, rows1)
  obufs = (ob0, ob1)
  gsems = (g0, g1)
  osems = (o0, o1)

  pltpu.async_copy(expert_hbm.at[idx_v.at[0]], rows0, g0)
  pltpu.async_copy(expert_hbm.at[idx_v.at[1]], rows1, g1)

  @pl.loop(0, _NCH, step=2)
  def _chunks(c):
    for b in range(2):
      ch = c + b
      pltpu.make_async_copy(
          expert_hbm.at[idx_v.at[ch]], rows[b], gsems[b]).wait()

      @pl.when(c >= 2)
      def _():
        pltpu.make_async_copy(
            obufs[b], out_hbm.at[pl.ds(tbase + (ch - 2) * _T, _T)],
            osems[b]).wait()

      for t in range(_T):
        w = [w_v[ch, t * _TK + k, :] for k in range(_TK)]

        @pl.loop(0, _SL)
        def _sloop(s, _t=t, _w=w, _b=b):
          for j in range(_LN // 16):
            sl = pl.ds(j * 16, 16)
            acc = _w[0] * plsc.bitcast(
                rows[_b][_t * _TK, s, sl], jnp.bfloat16)
            for k in range(1, _TK):
              acc = acc + _w[k] * plsc.bitcast(
                  rows[_b][_t * _TK + k, s, sl], jnp.bfloat16)
            obufs[_b][_t, s, sl] = plsc.bitcast(acc, jnp.int32)

      pltpu.async_copy(
          obufs[b], out_hbm.at[pl.ds(tbase + ch * _T, _T)], osems[b])

      @pl.when(ch + 2 < _NCH)
      def _():
        pltpu.async_copy(
            expert_hbm.at[idx_v.at[ch + 2]], rows[b], gsems[b])

  for b in range(2):
    pltpu.make_async_copy(
        obufs[b], out_hbm.at[pl.ds(tbase + (_NCH - 2 + b) * _T, _T)],
        osems[b]).wait()


def kernel(expert_output, topk_vals, inv_perm):
  expert_i32 = jax.lax.bitcast_convert_type(
      expert_output.reshape(_TE, _HW, 2), jnp.int32).reshape(_TE, _SL, _LN)
  inv = inv_perm.astype(jnp.int32).reshape(_NW, _NCH, _RPC)
  wv = jnp.broadcast_to(
      topk_vals.astype(jnp.bfloat16).reshape(_NW, _NCH, _RPC, 1),
      (_NW, _NCH, _RPC, 32))
  out_i32 = _combine(expert_i32, inv, wv)
  out = jax.lax.bitcast_convert_type(
      out_i32.reshape(_NT, _HW), jnp.bfloat16)
  return out.reshape(_NT, _H)


# speedup vs baseline: 2.1803x; 2.1803x over previous
import functools

import jax
import jax.numpy as jnp
from jax import lax
from jax.experimental import pallas as pl
from jax.experimental.pallas import tpu as pltpu
from jax.experimental.pallas import tpu_sc as plsc

_NT = 4096
_TK = 8
_H = 4096
_TE = _NT * _TK
_NC = 2
_NS = 16
_NW = _NC * _NS
_TPW = _NT // _NW
_EPW = _TPW * _TK
_T = 2
_RPC = _T * _TK
_NCH = _TPW // _T

_mesh = plsc.VectorSubcoreMesh(core_axis_name="c", subcore_axis_name="s")


@functools.partial(
    pl.kernel,
    out_type=jax.ShapeDtypeStruct((_NT, _H), jnp.bfloat16),
    mesh=_mesh,
    compiler_params=pltpu.CompilerParams(
        use_tc_tiling_on_sc=False, needs_layout_passes=False),
    scratch_types=[
        pltpu.VMEM((_EPW,), jnp.int32),
        pltpu.VMEM((_NCH, _RPC, 32), jnp.bfloat16),
        pltpu.VMEM((_RPC, _H), jnp.bfloat16),
        pltpu.VMEM((_RPC, _H), jnp.bfloat16),
        pltpu.VMEM((_T, _H), jnp.bfloat16),
        pltpu.VMEM((_T, _H), jnp.bfloat16),
        pltpu.SemaphoreType.DMA,
        pltpu.SemaphoreType.DMA,
        pltpu.SemaphoreType.DMA,
        pltpu.SemaphoreType.DMA,
    ],
)
def _combine(expert_hbm, inv_hbm, w_hbm, out_hbm,
             idx_v, w_v, rows0, rows1, ob0, ob1, g0, g1, o0, o1):
  wid = lax.axis_index("s") * _NC + lax.axis_index("c")
  tbase = wid * _TPW

  pltpu.sync_copy(inv_hbm.at[pl.ds(wid * _EPW, _EPW)], idx_v)
  pltpu.sync_copy(w_hbm.at[wid], w_v)

  rows = (rows0, rows1)
  obufs = (ob0, ob1)
  gsems = (g0, g1)
  osems = (o0, o1)

  pltpu.async_copy(expert_hbm.at[idx_v.at[pl.ds(0, _RPC)]], rows0, g0)
  pltpu.async_copy(expert_hbm.at[idx_v.at[pl.ds(_RPC, _RPC)]], rows1, g1)

  @pl.loop(0, _NCH, step=2)
  def _chunks(c):
    for b in range(2):
      ch = c + b
      pltpu.make_async_copy(
          expert_hbm.at[idx_v.at[pl.ds(ch * _RPC, _RPC)]], rows[b],
          gsems[b]).wait()

      @pl.when(c >= 2)
      def _():
        pltpu.make_async_copy(
            obufs[b], out_hbm.at[pl.ds(tbase + (ch - 2) * _T, _T)],
            osems[b]).wait()

      for t in range(_T):
        w = [w_v[ch, t * _TK + k, :] for k in range(_TK)]

        @pl.loop(0, _H // 32)
        def _sloop(s, _t=t, _w=w, _b=b):
          sl = pl.ds(pl.multiple_of(s * 32, 32), 32)
          acc = _w[0] * rows[_b][_t * _TK, sl]
          for k in range(1, _TK):
            acc = acc + _w[k] * rows[_b][_t * _TK + k, sl]
          obufs[_b][_t, sl] = acc

      pltpu.async_copy(
          obufs[b], out_hbm.at[pl.ds(tbase + ch * _T, _T)], osems[b])

      @pl.when(ch + 2 < _NCH)
      def _():
        pltpu.async_copy(
            expert_hbm.at[idx_v.at[pl.ds((ch + 2) * _RPC, _RPC)]],
            rows[b], gsems[b])

  for b in range(2):
    pltpu.make_async_copy(
        obufs[b], out_hbm.at[pl.ds(tbase + (_NCH - 2 + b) * _T, _T)],
        osems[b]).wait()


def kernel(expert_output, topk_vals, inv_perm):
  inv = inv_perm.astype(jnp.int32)
  wv = jnp.broadcast_to(
      topk_vals.astype(jnp.bfloat16).reshape(_NW, _NCH, _RPC, 1),
      (_NW, _NCH, _RPC, 32))
  return _combine(expert_output, inv, wv)


# speedup vs baseline: 5.3559x; 2.4565x over previous
import functools

import jax
import jax.numpy as jnp
from jax import lax
from jax.experimental import pallas as pl
from jax.experimental.pallas import tpu as pltpu
from jax.experimental.pallas import tpu_sc as plsc

_NT = 4096
_TK = 8
_H = 4096
_PW = _H
_TE = _NT * _TK
_NC = 2
_NS = 16
_NW = _NC * _NS
_TPW = _NT // _NW
_EPW = _TPW * _TK
_NG = _PW // 16

_mesh = plsc.VectorSubcoreMesh(core_axis_name="c", subcore_axis_name="s")


@functools.partial(
    pl.kernel,
    out_type=jax.ShapeDtypeStruct((_NT, _H), jnp.bfloat16),
    mesh=_mesh,
    compiler_params=pltpu.CompilerParams(
        use_tc_tiling_on_sc=True, needs_layout_passes=False),
    scratch_types=[
        pltpu.VMEM((_EPW,), jnp.int32),
        pltpu.VMEM((2, _TK, 16), jnp.int32),
        pltpu.VMEM((_TK, _PW), jnp.int32),
        pltpu.VMEM((_TK, _PW), jnp.int32),
        pltpu.VMEM((_PW,), jnp.int32),
        pltpu.VMEM((_PW,), jnp.int32),
        pltpu.VMEM((_PW,), jnp.int32),
        pltpu.SemaphoreType.DMA,
        pltpu.SemaphoreType.DMA,
        pltpu.SemaphoreType.DMA,
        pltpu.SemaphoreType.DMA,
        pltpu.SemaphoreType.DMA,
        pltpu.SemaphoreType.DMA,
    ],
)
def _combine(expert_hbm, inv2_hbm, w_hbm, out_hbm,
             idx_v, w_v, rows0, rows1, ob0, ob1, stg,
             g0, g1, o0, o1, ws0, ws1):
  wid = lax.axis_index("s") * _NC + lax.axis_index("c")
  tbase = wid * _TPW
  expert_w = expert_hbm.bitcast(jnp.int32)
  out_w = out_hbm.bitcast(jnp.int32)

  pltpu.sync_copy(inv2_hbm.at[pl.ds(wid * _EPW, _EPW)], idx_v)

  rows = (rows0, rows1)
  obufs = (ob0, ob1)
  gsems = (g0, g1)
  osems = (o0, o1)
  wsems = (ws0, ws1)

  pltpu.async_copy(w_hbm.at[wid, 0], w_v.at[0], ws0)
  pltpu.async_copy(w_hbm.at[wid, 1], w_v.at[1], ws1)
  pltpu.async_copy(expert_w.at[idx_v.at[pl.ds(0, _TK)]], rows0, g0)
  pltpu.async_copy(expert_w.at[idx_v.at[pl.ds(_TK, _TK)]], rows1, g1)

  @pl.loop(0, _TPW, step=4)
  def _groups(c):
    for pb in range(2):
      for b in range(2):
        tok = c + pb * 2 + b
        pltpu.make_async_copy(
            expert_w.at[idx_v.at[pl.ds(tok * _TK, _TK)]], rows[b],
            gsems[b]).wait()
        pltpu.make_async_copy(
            w_hbm.at[wid, tok], w_v.at[b], wsems[b]).wait()

        if b == 1:
          @pl.when(c >= 4)
          def _():
            pltpu.make_async_copy(
                obufs[pb], out_w.at[(tbase + tok - 5) // 2],
                osems[pb]).wait()

        w = [plsc.bitcast(w_v[b, k, :], jnp.bfloat16) for k in range(_TK)]

        @pl.loop(0, _NG)
        def _sloop(s, _w=w, _b=b, _pb=pb):
          sl = pl.ds(pl.multiple_of(s * 16, 16), 16)
          acc = _w[0] * plsc.bitcast(rows[_b][0, sl], jnp.bfloat16)
          for k in range(1, _TK):
            acc = acc + _w[k] * plsc.bitcast(rows[_b][k, sl], jnp.bfloat16)
          if _b == 0:
            stg[sl] = plsc.bitcast(acc, jnp.int32)
          else:
            pe = plsc.bitcast(stg[sl], jnp.bfloat16)
            e0, e1 = plsc.unpack(pe, format=plsc.PackFormat.INTERLEAVED)
            q0, q1 = plsc.unpack(acc, format=plsc.PackFormat.INTERLEAVED)
            packed = plsc.pack(e0 + e1, q0 + q1,
                               format=plsc.PackFormat.INTERLEAVED)
            obufs[_pb][sl] = plsc.bitcast(packed, jnp.int32)

        if b == 1:
          pltpu.async_copy(
              obufs[pb], out_w.at[(tbase + tok - 1) // 2], osems[pb])

        @pl.when(tok + 2 < _TPW)
        def _():
          pltpu.async_copy(
              expert_w.at[idx_v.at[pl.ds((tok + 2) * _TK, _TK)]],
              rows[b], gsems[b])
          pltpu.async_copy(w_hbm.at[wid, tok + 2], w_v.at[b], wsems[b])

  for pb in range(2):
    pltpu.make_async_copy(
        obufs[pb], out_w.at[tbase // 2 + _TPW // 2 - 2 + pb],
        osems[pb]).wait()


def kernel(expert_output, topk_vals, inv_perm):
  inv = inv_perm.astype(jnp.int32)
  inv2 = inv // 2
  par = (inv & 1).reshape(_NW, _TPW, _TK, 1)
  wbits = jax.lax.bitcast_convert_type(
      topk_vals.astype(jnp.bfloat16).reshape(_NW, _TPW, _TK, 1, 1),
      jnp.uint16).astype(jnp.int32).reshape(_NW, _TPW, _TK, 1)
  wword = jnp.where(par == 0, wbits, wbits << 16)
  wv = jnp.broadcast_to(wword, (_NW, _TPW, _TK, 16))
  return _combine(expert_output, inv2, wv)


# speedup vs baseline: 10.4830x; 1.9573x over previous
import functools

import jax
import jax.numpy as jnp
from jax import lax
from jax.experimental import pallas as pl
from jax.experimental.pallas import tpu as pltpu
from jax.experimental.pallas import tpu_sc as plsc

_NT = 4096
_TK = 8
_H = 4096
_PW = _H
_TE = _NT * _TK
_NC = 2
_NS = 16
_NW = _NC * _NS
_TPW = _NT // _NW
_EPW = _TPW * _TK
_NG = _PW // 16

_mesh = plsc.VectorSubcoreMesh(core_axis_name="c", subcore_axis_name="s")


@functools.partial(
    pl.kernel,
    out_type=jax.ShapeDtypeStruct((_NT, _H), jnp.bfloat16),
    mesh=_mesh,
    compiler_params=pltpu.CompilerParams(
        use_tc_tiling_on_sc=True, needs_layout_passes=False),
    scratch_types=[
        pltpu.VMEM((_EPW,), jnp.int32),
        pltpu.VMEM((2, _TK, 16), jnp.int32),
        pltpu.VMEM((_TK, _PW), jnp.int32),
        pltpu.VMEM((_TK, _PW), jnp.int32),
        pltpu.VMEM((_PW,), jnp.int32),
        pltpu.VMEM((_PW,), jnp.int32),
        pltpu.VMEM((_PW,), jnp.int32),
        pltpu.SemaphoreType.DMA,
        pltpu.SemaphoreType.DMA,
        pltpu.SemaphoreType.DMA,
        pltpu.SemaphoreType.DMA,
        pltpu.SemaphoreType.DMA,
        pltpu.SemaphoreType.DMA,
    ],
)
def _combine(expert_hbm, inv2_hbm, w_hbm, out_hbm,
             idx_v, w_v, rows0, rows1, ob0, ob1, stg,
             g0, g1, o0, o1, ws0, ws1):
  wid = lax.axis_index("s") * _NC + lax.axis_index("c")
  tbase = wid * _TPW
  expert_w = expert_hbm.bitcast(jnp.int32)
  out_w = out_hbm.bitcast(jnp.int32)

  pltpu.sync_copy(inv2_hbm.at[pl.ds(wid * _EPW, _EPW)], idx_v)

  rows = (rows0, rows1)
  obufs = (ob0, ob1)
  gsems = (g0, g1)
  osems = (o0, o1)
  wsems = (ws0, ws1)

  pltpu.async_copy(w_hbm.at[wid, 0], w_v.at[0], ws0)
  pltpu.async_copy(w_hbm.at[wid, 1], w_v.at[1], ws1)
  pltpu.async_copy(expert_w.at[idx_v.at[pl.ds(0, _TK)]], rows0, g0)
  pltpu.async_copy(expert_w.at[idx_v.at[pl.ds(_TK, _TK)]], rows1, g1)

  @pl.loop(0, _TPW, step=4)
  def _groups(c):
    for pb in range(2):
      for b in range(2):
        tok = c + pb * 2 + b
        pltpu.make_async_copy(
            expert_w.at[idx_v.at[pl.ds(tok * _TK, _TK)]], rows[b],
            gsems[b]).wait()
        pltpu.make_async_copy(
            w_hbm.at[wid, tok], w_v.at[b], wsems[b]).wait()

        if b == 1:
          @pl.when(c >= 4)
          def _():
            pltpu.make_async_copy(
                obufs[pb], out_w.at[(tbase + tok - 5) // 2],
                osems[pb]).wait()

        w = [plsc.bitcast(w_v[b, k, :], jnp.bfloat16) for k in range(_TK)]

        @plsc.parallel_loop(0, _NG, unroll=2)
        def _sloop(s, _w=w, _b=b, _pb=pb):
          sl = pl.ds(pl.multiple_of(s * 16, 16), 16)
          acc = _w[0] * plsc.bitcast(rows[_b][0, sl], jnp.bfloat16)
          for k in range(1, _TK):
            acc = acc + _w[k] * plsc.bitcast(rows[_b][k, sl], jnp.bfloat16)
          if _b == 0:
            stg[sl] = plsc.bitcast(acc, jnp.int32)
          else:
            pe = plsc.bitcast(stg[sl], jnp.bfloat16)
            e0, e1 = plsc.unpack(pe, format=plsc.PackFormat.INTERLEAVED)
            q0, q1 = plsc.unpack(acc, format=plsc.PackFormat.INTERLEAVED)
            packed = plsc.pack(e0 + e1, q0 + q1,
                               format=plsc.PackFormat.INTERLEAVED)
            obufs[_pb][sl] = plsc.bitcast(packed, jnp.int32)

        if b == 1:
          pltpu.async_copy(
              obufs[pb], out_w.at[(tbase + tok - 1) // 2], osems[pb])

        @pl.when(tok + 2 < _TPW)
        def _():
          pltpu.async_copy(
              expert_w.at[idx_v.at[pl.ds((tok + 2) * _TK, _TK)]],
              rows[b], gsems[b])
          pltpu.async_copy(w_hbm.at[wid, tok + 2], w_v.at[b], wsems[b])

  for pb in range(2):
    pltpu.make_async_copy(
        obufs[pb], out_w.at[tbase // 2 + _TPW // 2 - 2 + pb],
        osems[pb]).wait()


def kernel(expert_output, topk_vals, inv_perm):
  inv = inv_perm.astype(jnp.int32)
  inv2 = inv // 2
  par = (inv & 1).reshape(_NW, _TPW, _TK, 1)
  wbits = jax.lax.bitcast_convert_type(
      topk_vals.astype(jnp.bfloat16).reshape(_NW, _TPW, _TK, 1, 1),
      jnp.uint16).astype(jnp.int32).reshape(_NW, _TPW, _TK, 1)
  wword = jnp.where(par == 0, wbits, wbits << 16)
  wv = jnp.broadcast_to(wword, (_NW, _TPW, _TK, 16))
  return _combine(expert_output, inv2, wv)
